# Initial kernel scaffold; baseline (speedup 1.0000x reference)
import functools

import jax
import jax.numpy as jnp
from jax import lax
from jax.experimental import pallas as pl
from jax.experimental.pallas import tpu as pltpu
from jax.experimental.pallas import tpu_sc as plsc

_L = 16


def _make_sc_spmm(N, B, K, T):
    mesh = plsc.VectorSubcoreMesh(core_axis_name="c", subcore_axis_name="s")
    n_sub = 16
    rows_per_tile = N // n_sub

    @functools.partial(
        pl.kernel,
        mesh=mesh,
        out_type=jax.ShapeDtypeStruct((2 * N, B), jnp.float32),
        scratch_types=[
            pltpu.VMEM((K,), jnp.int32),
            pltpu.VMEM((K,), jnp.int32),
            pltpu.VMEM((K,), jnp.float32),
            pltpu.VMEM((K, B), jnp.float32),
            pltpu.VMEM_SHARED((N, B), jnp.float32),
            pltpu.SemaphoreType.DMA,
        ],
    )
    def sc_spmm(xT_hbm, col_hbm, row_hbm, val_hbm, out_hbm,
                colv, rowv, valv, rows, acc, sem):
        c = lax.axis_index("c")
        s = lax.axis_index("s")
        wid = s * 2 + c

        zero = jnp.zeros((_L,), jnp.float32)

        def zero_body(j, carry):
            for cc in range(B // _L):
                rows[j, pl.ds(cc * _L, _L)] = zero
            return carry

        lax.fori_loop(0, K, zero_body, 0)
        for k in range(rows_per_tile // K):
            pltpu.sync_copy(rows, acc.at[pl.ds(s * rows_per_tile + k * K, K)])
        plsc.subcore_barrier()

        base = wid * (K * T)

        def batch_body(t, carry):
            off = base + t * K
            pltpu.sync_copy(col_hbm.at[pl.ds(off, K)], colv)
            pltpu.sync_copy(row_hbm.at[pl.ds(off, K)], rowv)
            pltpu.sync_copy(val_hbm.at[pl.ds(off, K)], valv)
            pltpu.async_copy(xT_hbm.at[colv], rows, sem).wait()

            def scale_body(j, inner):
                v = plsc.load_gather(valv, [jnp.full((_L,), j, jnp.int32)])
                for cc in range(B // _L):
                    blk = rows[j, pl.ds(cc * _L, _L)]
                    rows[j, pl.ds(cc * _L, _L)] = blk * v
                return inner

            lax.fori_loop(0, K, scale_body, 0)
            pltpu.sync_copy(rows, acc.at[rowv], add=True)
            return carry

        lax.fori_loop(0, T, batch_body, 0)
        plsc.subcore_barrier()

        for k in range(rows_per_tile // K):
            r0 = s * rows_per_tile + k * K
            pltpu.sync_copy(acc.at[pl.ds(r0, K)], rows)
            pltpu.sync_copy(rows, out_hbm.at[pl.ds(c * N + r0, K)])

    return sc_spmm


def kernel(x, values, row_ids, col_idx, row_offs):
    B, N = x.shape
    NNZ = values.shape[0]
    K = 128
    NW = 32
    T = -(-NNZ // (NW * K))
    pad = NW * K * T - NNZ

    xT = x.T
    colp = jnp.concatenate([col_idx, jnp.zeros((pad,), jnp.int32)])
    rowp = jnp.concatenate([row_ids, jnp.zeros((pad,), jnp.int32)])
    valp = jnp.concatenate([values, jnp.zeros((pad,), values.dtype)])

    partials = _make_sc_spmm(N, B, K, T)(xT, colp, rowp, valp)
    return (partials[:N] + partials[N:]).T


# speedup vs baseline: 1.2882x; 1.2882x over previous
import functools

import jax
import jax.numpy as jnp
from jax import lax
from jax.experimental import pallas as pl
from jax.experimental.pallas import tpu as pltpu
from jax.experimental.pallas import tpu_sc as plsc

_L = 16


def _make_sc_spmm(N, B, K):
    mesh = plsc.VectorSubcoreMesh(core_axis_name="c", subcore_axis_name="s")
    NW = 32
    R = N // NW

    @functools.partial(
        pl.kernel,
        mesh=mesh,
        out_type=jax.ShapeDtypeStruct((N, B), jnp.float32),
        compiler_params=pltpu.CompilerParams(needs_layout_passes=False),
        scratch_types=[
            pltpu.VMEM((16,), jnp.int32),
            pltpu.VMEM((16,), jnp.int32),
            pltpu.VMEM((K,), jnp.int32),
            pltpu.VMEM((K, _L), jnp.int32),
            pltpu.VMEM((K, _L), jnp.float32),
            pltpu.VMEM((K, B), jnp.float32),
            pltpu.VMEM((R, B), jnp.float32),
            pltpu.SemaphoreType.DMA,
        ],
    )
    def sc_spmm(xT_hbm, col_hbm, row_hbm, val_hbm, offs_hbm, out_hbm,
                ov0, ov1, colv, rowv, valv, rows, acc, sem):
        c = lax.axis_index("c")
        s = lax.axis_index("s")
        wid = s * 2 + c
        r0 = wid * R

        pltpu.sync_copy(offs_hbm.at[pl.ds(r0, 16)], ov0)
        pltpu.sync_copy(offs_hbm.at[pl.ds(r0 + R, 16)], ov1)
        s0 = jnp.min(ov0[pl.ds(0, _L)])
        e0 = jnp.min(ov1[pl.ds(0, _L)])
        a0 = (s0 >> 3) << 3
        nb = (e0 - a0 + (K - 1)) >> 7

        zero = jnp.zeros((_L,), jnp.float32)

        def zero_body(r, carry):
            for cc in range(B // _L):
                acc[r, pl.ds(cc * _L, _L)] = zero
            return carry

        lax.fori_loop(0, R, zero_body, 0)

        iota = lax.iota(jnp.int32, _L)

        def batch_body(t, carry):
            off = pl.multiple_of(a0 + t * K, 8)
            pltpu.sync_copy(col_hbm.at[pl.ds(off, K)], colv)
            pltpu.sync_copy(row_hbm.at[pl.ds(off, K), :], rowv)
            pltpu.sync_copy(val_hbm.at[pl.ds(off, K), :], valv)
            pltpu.async_copy(xT_hbm.at[colv], rows, sem).wait()

            U = 8

            def nnz_body(u, inner):
                for du in range(U):
                    j = u * U + du
                    g = off + j
                    valid = jnp.logical_and(g >= s0, g < e0)
                    w = jnp.where(valid, jnp.float32(1.0), jnp.float32(0.0))
                    v = valv[j, pl.ds(0, _L)] * w
                    rloc = jnp.clip(rowv[j, pl.ds(0, _L)] - r0, 0, R - 1)
                    for cc in range(B // _L):
                        xb = rows[j, pl.ds(cc * _L, _L)]
                        plsc.addupdate_scatter(
                            acc, [rloc, iota + cc * _L], xb * v)
                return inner

            lax.fori_loop(0, K // U, nnz_body, 0)
            return carry

        lax.fori_loop(0, nb, batch_body, 0)

        pltpu.sync_copy(acc, out_hbm.at[pl.ds(r0, R)])

    return sc_spmm


def kernel(x, values, row_ids, col_idx, row_offs):
    B, N = x.shape
    NNZ = values.shape[0]
    K = 128

    xT = x.T
    pad = K + 8
    colp = jnp.concatenate([col_idx, jnp.zeros((pad,), jnp.int32)])
    rowp = jnp.concatenate([row_ids, jnp.zeros((pad,), jnp.int32)])
    valp = jnp.concatenate([values, jnp.zeros((pad,), values.dtype)])
    val16 = jnp.broadcast_to(valp[:, None], (valp.shape[0], _L))
    row16 = jnp.broadcast_to(rowp[:, None], (rowp.shape[0], _L))
    offsp = jnp.concatenate([row_offs, jnp.full((15,), NNZ, jnp.int32)])

    outT = _make_sc_spmm(N, B, K)(xT, colp, row16, val16, offsp)
    return outT.T


# speedup vs baseline: 2.0810x; 1.6155x over previous
import functools

import jax
import jax.numpy as jnp
from jax import lax
from jax.experimental import pallas as pl
from jax.experimental.pallas import tpu as pltpu
from jax.experimental.pallas import tpu_sc as plsc

_L = 16


def _make_sc_spmm(N, B, K):
    mesh = plsc.VectorSubcoreMesh(core_axis_name="c", subcore_axis_name="s")
    NW = 32
    R = N // NW

    @functools.partial(
        pl.kernel,
        mesh=mesh,
        out_type=jax.ShapeDtypeStruct((N * B,), jnp.float32),
        compiler_params=pltpu.CompilerParams(needs_layout_passes=False),
        scratch_types=[
            pltpu.VMEM((16,), jnp.int32),
            pltpu.VMEM((16,), jnp.int32),
            pltpu.VMEM((K,), jnp.int32),
            pltpu.VMEM((K, _L), jnp.int32),
            pltpu.VMEM((K, _L), jnp.float32),
            pltpu.VMEM((K, B), jnp.float32),
            pltpu.VMEM((R * B,), jnp.float32),
            pltpu.SemaphoreType.DMA,
        ],
    )
    def sc_spmm(xT_hbm, col_hbm, adr_hbm, val_hbm, offs_hbm, out_hbm,
                ov0, ov1, colv, adrv, valv, rows, acc, sem):
        c = lax.axis_index("c")
        s = lax.axis_index("s")
        wid = s * 2 + c
        r0 = wid * R

        pltpu.sync_copy(offs_hbm.at[pl.ds(r0, 16)], ov0)
        pltpu.sync_copy(offs_hbm.at[pl.ds(r0 + R, 16)], ov1)
        s0 = jnp.min(ov0[pl.ds(0, _L)])
        e0 = jnp.min(ov1[pl.ds(0, _L)])
        a0 = (s0 >> 3) << 3
        nb = (e0 - a0 + (K - 1)) >> 7

        zero = jnp.zeros((_L,), jnp.float32)

        @plsc.parallel_loop(0, R, 1, unroll=4)
        def _(r):
            for cc in range(B // _L):
                acc[pl.ds(r * B + cc * _L, _L)] = zero

        iota = lax.iota(jnp.int32, _L)
        abase = r0 * B
        alo = iota
        ahi = iota + (R - 1) * B

        def do_batch(t, masked):
            off = pl.multiple_of(a0 + t * K, 8)
            pltpu.sync_copy(col_hbm.at[pl.ds(off, K)], colv)
            pltpu.sync_copy(adr_hbm.at[pl.ds(off, K), :], adrv)
            pltpu.sync_copy(val_hbm.at[pl.ds(off, K), :], valv)
            pltpu.async_copy(xT_hbm.at[colv], rows, sem).wait()

            @plsc.parallel_loop(0, K, 1, unroll=4)
            def _(j):
                v = valv[j, pl.ds(0, _L)]
                a = adrv[j, pl.ds(0, _L)] - abase
                if masked:
                    g = off + j
                    valid = jnp.logical_and(g >= s0, g < e0)
                    w = jnp.where(valid, jnp.float32(1.0), jnp.float32(0.0))
                    v = v * w
                    a = jnp.minimum(jnp.maximum(a, alo), ahi)
                for cc in range(B // _L):
                    xb = rows[j, pl.ds(cc * _L, _L)]
                    plsc.addupdate_scatter(acc, [a + cc * _L], xb * v)

        @pl.when(nb > 0)
        def _():
            do_batch(0, True)

        @pl.when(nb > 1)
        def _():
            do_batch(nb - 1, True)

        def interior(t, carry):
            do_batch(t, False)
            return carry

        lax.fori_loop(1, nb - 1, interior, 0)

        pltpu.sync_copy(acc, out_hbm.at[pl.ds(r0 * B, R * B)])

    return sc_spmm


def kernel(x, values, row_ids, col_idx, row_offs):
    B, N = x.shape
    NNZ = values.shape[0]
    K = 128

    xT = x.T
    pad = K + 8
    colp = jnp.concatenate([col_idx, jnp.zeros((pad,), jnp.int32)])
    rowp = jnp.concatenate([row_ids, jnp.zeros((pad,), jnp.int32)])
    valp = jnp.concatenate([values, jnp.zeros((pad,), values.dtype)])
    lanes = jnp.arange(_L, dtype=jnp.int32)
    adr16 = rowp[:, None] * B + lanes[None, :]
    val16 = jnp.broadcast_to(valp[:, None], (valp.shape[0], _L))
    offsp = jnp.concatenate([row_offs, jnp.full((15,), NNZ, jnp.int32)])

    outT = _make_sc_spmm(N, B, K)(xT, colp, adr16, val16, offsp)
    return outT.reshape(N, B).T
